# baseline (speedup 1.0000x reference)
import functools

import jax
import jax.numpy as jnp
from jax import lax
from jax.experimental import pallas as pl
from jax.experimental.pallas import tpu as pltpu
from jax.experimental.pallas import tpu_sc as plsc

B = 16384
D = 16
L = 16
NC = 2
NS = 16
NW = NC * NS
BPW = B // NW
NCHUNK = 4
CHUNK = BPW // NCHUNK


def _sc_body(user_hbm, skill_hbm, ut_hbm, st_hbm, wb_hbm, out_hbm,
             idx_u, idx_s, rows_u, rows_s, out_v, wv, sem):
    wid = lax.axis_index("s") * NC + lax.axis_index("c")
    base = wid * BPW

    pltpu.sync_copy(wb_hbm, wv)
    pltpu.sync_copy(user_hbm.at[wid], idx_u)
    pltpu.sync_copy(skill_hbm.at[wid], idx_s)

    copies = []
    for j in range(NCHUNK):
        copies.append(pltpu.async_copy(
            ut_hbm.at[idx_u.at[j]], rows_u.at[pl.ds(j * CHUNK, CHUNK)], sem))
        copies.append(pltpu.async_copy(
            st_hbm.at[idx_s.at[j]], rows_s.at[pl.ds(j * CHUNK, CHUNK)], sem))
    for c in copies:
        c.wait()

    lane = lax.iota(jnp.int32, L)
    w_u = wv[0]
    w_s = wv[1]
    bb = wv[2]

    def xperm(x, s):
        return jnp.take_along_axis(x, lane ^ s, axis=0,
                                   mode="promise_in_bounds")

    def combine(x, y, s):
        return jnp.where((lane & s) == 0, x + xperm(x, s), y + xperm(y, s))

    def group(g, carry):
        ps = [rows_u[g * L + j] * w_u + rows_s[g * L + j] * w_s
              for j in range(L)]
        ps = [combine(ps[i], ps[i + 8], 8) for i in range(8)]
        ps = [combine(ps[i], ps[i + 4], 4) for i in range(4)]
        ps = [combine(ps[i], ps[i + 2], 2) for i in range(2)]
        acc = combine(ps[0], ps[1], 1) + bb
        out_v[pl.ds(g * L, L)] = acc
        return carry

    lax.fori_loop(0, BPW // L, group, 0)

    pltpu.sync_copy(out_v, out_hbm.at[pl.ds(base, BPW)])


@functools.partial(
    pl.kernel,
    out_type=jax.ShapeDtypeStruct((B,), jnp.float32),
    mesh=plsc.VectorSubcoreMesh(core_axis_name="c", subcore_axis_name="s"),
    compiler_params=pltpu.CompilerParams(use_tc_tiling_on_sc=False),
    scratch_types=[
        pltpu.VMEM((NCHUNK, CHUNK), jnp.int32),
        pltpu.VMEM((NCHUNK, CHUNK), jnp.int32),
        pltpu.VMEM((BPW, D), jnp.float32),
        pltpu.VMEM((BPW, D), jnp.float32),
        pltpu.VMEM((BPW,), jnp.float32),
        pltpu.VMEM((3, L), jnp.float32),
        pltpu.SemaphoreType.DMA,
    ],
)
def _sc_kernel(user_hbm, skill_hbm, ut_hbm, st_hbm, wb_hbm, out_hbm,
               idx_u, idx_s, rows_u, rows_s, out_v, wv, sem):
    _sc_body(user_hbm, skill_hbm, ut_hbm, st_hbm, wb_hbm, out_hbm,
             idx_u, idx_s, rows_u, rows_s, out_v, wv, sem)


def kernel(user, skill, user_table, skill_table, W, b):
    user_r = user.astype(jnp.int32).reshape(NW, NCHUNK, CHUNK)
    skill_r = skill.astype(jnp.int32).reshape(NW, NCHUNK, CHUNK)
    wb = jnp.stack(
        [W[:D, 0], W[D:, 0], jnp.broadcast_to(b.astype(jnp.float32), (L,))]
    ).astype(jnp.float32)
    return _sc_kernel(user_r, skill_r, user_table, skill_table, wb)
